# Initial kernel scaffold; baseline (speedup 1.0000x reference)
import jax
import jax.numpy as jnp
from jax.experimental import pallas as pl


def kernel(x, edge_index, edge_attr, batch, task_id, c0_w1, c0_b1, c0_w2, c0_b2, c0_root, c0_bias, c1_w1, c1_b1, c1_w2, c1_b2, c1_root, c1_bias, lin0_w, lin0_b, lin1_w, lin1_b, out_w, out_b):
    raise NotImplementedError("write your pallas kernel here")


# speedup vs baseline: 1.6436x; 1.6436x over previous
import functools

import jax
import jax.numpy as jnp
from jax import lax
from jax.experimental import pallas as pl
from jax.experimental.pallas import tpu as pltpu
from jax.experimental.pallas import tpu_sc as plsc

N = 10000
E = 160000
D_FEAT = 128
CONV_H = 8
N_GRAPHS = 64

NP = 10240
EP = 163840
K = 34
SC = K * CONV_H
BLK_E = 1024
BLK_N = 1024
PAD_DST = 10016


def _edge_body(ea_ref, xg_ref, w1_ref, b1_ref, wall_ref, r32_ref, bvec_ref,
               c_ref, msg_ref):
    t = jnp.dot(ea_ref[...], w1_ref[...], preferred_element_type=jnp.float32)
    t = t + b1_ref[...]
    h = jnp.where(t > 0, t, 0.01 * t)
    hexp = jnp.dot(h, r32_ref[...], preferred_element_type=jnp.float32)
    hexp = hexp + bvec_ref[...]
    sg = jnp.dot(xg_ref[...], wall_ref[...], preferred_element_type=jnp.float32)
    msg_ref[...] = jnp.dot(hexp * sg, c_ref[...],
                           preferred_element_type=jnp.float32)


def _edge_stage(ea, xg, w1, b1, wall, r32, bvec, cmat):
    din = xg.shape[1]
    grid = EP // BLK_E
    return pl.pallas_call(
        _edge_body,
        grid=(grid,),
        in_specs=[
            pl.BlockSpec((BLK_E, 4), lambda i: (i, 0)),
            pl.BlockSpec((BLK_E, din), lambda i: (i, 0)),
            pl.BlockSpec((4, 32), lambda i: (0, 0)),
            pl.BlockSpec((1, 32), lambda i: (0, 0)),
            pl.BlockSpec((din, SC), lambda i: (0, 0)),
            pl.BlockSpec((32, SC), lambda i: (0, 0)),
            pl.BlockSpec((1, SC), lambda i: (0, 0)),
            pl.BlockSpec((SC, 128), lambda i: (0, 0)),
        ],
        out_specs=pl.BlockSpec((BLK_E, 128), lambda i: (i, 0)),
        out_shape=jax.ShapeDtypeStruct((EP, 128), jnp.float32),
    )(ea, xg, w1, b1, wall, r32, bvec, cmat)


def _node0_body(p_ref, x_ref, root_ref, bias_ref, h1_ref):
    aggr = p_ref[0, :, :8] + p_ref[1, :, :8]
    r = jnp.dot(x_ref[...], root_ref[...], preferred_element_type=jnp.float32)
    h1 = jnp.maximum(aggr + r + bias_ref[...], 0.0)
    h1_ref[...] = jnp.pad(h1, ((0, 0), (0, 120)))


def _node0_stage(p, x, root, bias):
    grid = NP // BLK_N
    return pl.pallas_call(
        _node0_body,
        grid=(grid,),
        in_specs=[
            pl.BlockSpec((2, BLK_N, 128), lambda i: (0, i, 0)),
            pl.BlockSpec((BLK_N, D_FEAT), lambda i: (i, 0)),
            pl.BlockSpec((D_FEAT, 8), lambda i: (0, 0)),
            pl.BlockSpec((1, 8), lambda i: (0, 0)),
        ],
        out_specs=pl.BlockSpec((BLK_N, 128), lambda i: (i, 0)),
        out_shape=jax.ShapeDtypeStruct((NP, 128), jnp.float32),
    )(p, x, root, bias)


def _node1_body(p_ref, h1_ref, root_ref, bias_ref, batch_ref, pooled_ref):
    i = pl.program_id(0)

    @pl.when(i == 0)
    def _():
        pooled_ref[...] = jnp.zeros_like(pooled_ref)

    aggr = p_ref[0, :, :8] + p_ref[1, :, :8]
    r = jnp.dot(h1_ref[...], root_ref[...], preferred_element_type=jnp.float32)
    h2 = jnp.maximum(aggr + r + bias_ref[...], 0.0)
    g = batch_ref[0, 0, :]
    onehot = (g[None, :] == lax.broadcasted_iota(jnp.int32, (N_GRAPHS, BLK_N), 0)
              ).astype(jnp.float32)
    pooled_ref[...] += jnp.dot(onehot, h2, preferred_element_type=jnp.float32)


def _node1_stage(p, h1, root_pad, bias, batch3d):
    grid = NP // BLK_N
    return pl.pallas_call(
        _node1_body,
        grid=(grid,),
        in_specs=[
            pl.BlockSpec((2, BLK_N, 128), lambda i: (0, i, 0)),
            pl.BlockSpec((BLK_N, 128), lambda i: (i, 0)),
            pl.BlockSpec((128, 8), lambda i: (0, 0)),
            pl.BlockSpec((1, 8), lambda i: (0, 0)),
            pl.BlockSpec((1, 1, BLK_N), lambda i: (i, 0, 0)),
        ],
        out_specs=pl.BlockSpec((N_GRAPHS, 8), lambda i: (0, 0)),
        out_shape=jax.ShapeDtypeStruct((N_GRAPHS, 8), jnp.float32),
    )(p, h1, root_pad, bias, batch3d)


def _head_body(pooled_ref, w0_ref, b0_ref, w1_ref, b1_ref, w2_ref, b2_ref,
               out_ref):
    h = jnp.maximum(jnp.dot(pooled_ref[...], w0_ref[...],
                            preferred_element_type=jnp.float32) + b0_ref[...], 0.0)
    h = jnp.maximum(jnp.dot(h, w1_ref[...],
                            preferred_element_type=jnp.float32) + b1_ref[...], 0.0)
    out_ref[...] = jnp.dot(h, w2_ref[...],
                           preferred_element_type=jnp.float32) + b2_ref[...]


def _head_stage(pooled, w0, b0, w1, b1, w2, b2):
    return pl.pallas_call(
        _head_body,
        out_shape=jax.ShapeDtypeStruct((N_GRAPHS, 12), jnp.float32),
    )(pooled, w0, b0, w1, b1, w2, b2)


def _build_wall(w2, b2, in_ch):
    w2r = jnp.transpose(w2.reshape(32, in_ch, CONV_H), (1, 0, 2))
    wall = jnp.concatenate([
        w2r.reshape(in_ch, 32 * CONV_H),
        b2.reshape(in_ch, CONV_H),
        jnp.zeros((in_ch, CONV_H), jnp.float32),
    ], axis=1)
    return wall


def kernel(x, edge_index, edge_attr, batch, task_id, c0_w1, c0_b1, c0_w2,
           c0_b2, c0_root, c0_bias, c1_w1, c1_b1, c1_w2, c1_b2, c1_root,
           c1_bias, lin0_w, lin0_b, lin1_w, lin1_b, out_w, out_b):
    f32 = jnp.float32
    src = edge_index[0]
    dst = edge_index[1]
    src_p = jnp.pad(src, (0, EP - E))
    dst_p = jnp.pad(dst, (0, EP - E), constant_values=PAD_DST)
    ea_p = jnp.pad(edge_attr, ((0, EP - E), (0, 0)))
    x_p = jnp.pad(x, ((0, NP - N), (0, 0)))
    batch_p = jnp.pad(batch, (0, NP - N), constant_values=127).reshape(
        NP // BLK_N, 1, BLK_N)

    kk = jnp.arange(K * CONV_H, dtype=jnp.int32)
    r32 = (kk[None, :] // CONV_H == jnp.arange(32, dtype=jnp.int32)[:, None]
           ).astype(f32)
    bvec = (kk // CONV_H == 32).astype(f32)[None, :]
    cmat = (kk[:, None] % CONV_H ==
            jnp.arange(128, dtype=jnp.int32)[None, :]).astype(f32)
    cmat = cmat * (jnp.arange(128)[None, :] < 8).astype(f32)

    wall0 = _build_wall(c0_w2, c0_b2, D_FEAT)
    wall1 = jnp.pad(_build_wall(c1_w2, c1_b2, CONV_H), ((0, 120), (0, 0)))
    root1 = jnp.pad(c1_root, ((0, 120), (0, 0)))

    b1_0 = c0_b1.reshape(1, 32)
    b1_1 = c1_b1.reshape(1, 32)
    bias0 = c0_bias.reshape(1, 8)
    bias1 = c1_bias.reshape(1, 8)

    xg = _sc_gather(x_p, src_p)
    msg0 = _edge_stage(ea_p, xg, c0_w1, b1_0, wall0, r32, bvec, cmat)
    p0 = _sc_scatter(msg0, dst_p)
    h1 = _node0_stage(p0, x_p, c0_root, bias0)

    h1g = _sc_gather(h1, src_p)
    msg1 = _edge_stage(ea_p, h1g, c1_w1, b1_1, wall1, r32, bvec, cmat)
    p1 = _sc_scatter(msg1, dst_p)
    pooled = _node1_stage(p1, h1, root1, bias1, batch_p)

    return _head_stage(pooled, lin0_w, lin0_b.reshape(1, 64), lin1_w,
                       lin1_b.reshape(1, 64), out_w, out_b.reshape(1, 12))


NW = 32
CHUNK = 128
PER_W = EP // NW
N_CHUNKS = PER_W // CHUNK


def _sc_gather(table, idx):
    d = table.shape[1]
    mesh = plsc.VectorSubcoreMesh(core_axis_name="c", subcore_axis_name="s")

    @functools.partial(
        pl.kernel, mesh=mesh,
        out_type=jax.ShapeDtypeStruct((EP, d), jnp.float32),
        scratch_types=[
            pltpu.VMEM((CHUNK,), jnp.int32),
            pltpu.VMEM((CHUNK, d), jnp.float32),
            pltpu.SemaphoreType.DMA,
        ],
    )
    def k(table_hbm, idx_hbm, out_hbm, idx_v, rows_v, sem):
        wid = lax.axis_index("s") * 2 + lax.axis_index("c")
        base = wid * PER_W

        def body(c, carry):
            off = base + c * CHUNK
            pltpu.sync_copy(idx_hbm.at[pl.ds(off, CHUNK)], idx_v)
            pltpu.async_copy(table_hbm.at[idx_v], rows_v, sem).wait()
            pltpu.sync_copy(rows_v, out_hbm.at[pl.ds(off, CHUNK)])
            return carry

        lax.fori_loop(0, N_CHUNKS, body, 0)

    return k(table, idx)


ROWS_PER_S = NP // 16


def _sc_scatter(msg, dst):
    mesh = plsc.VectorSubcoreMesh(core_axis_name="c", subcore_axis_name="s")

    @functools.partial(
        pl.kernel, mesh=mesh,
        out_type=jax.ShapeDtypeStruct((2, NP, 128), jnp.float32),
        scratch_types=[
            pltpu.VMEM((CHUNK,), jnp.int32),
            pltpu.VMEM((CHUNK, 128), jnp.float32),
            pltpu.VMEM((CHUNK, 128), jnp.float32),
            pltpu.VMEM_SHARED((NP, 128), jnp.float32),
        ],
    )
    def k(msg_hbm, dst_hbm, out_hbm, idx_v, buf_v, zero_v, acc):
        c = lax.axis_index("c")
        s = lax.axis_index("s")
        wid = s * 2 + c
        base = wid * PER_W

        def zvec(i, carry):
            zero_v[i // 8, pl.ds((i % 8) * 16, 16)] = jnp.zeros((16,),
                                                               jnp.float32)
            return carry

        lax.fori_loop(0, CHUNK * 8, zvec, 0)

        def zcp(j, carry):
            pltpu.sync_copy(zero_v, acc.at[pl.ds(s * ROWS_PER_S + j * CHUNK,
                                                 CHUNK)])
            return carry

        lax.fori_loop(0, ROWS_PER_S // CHUNK, zcp, 0)
        plsc.subcore_barrier()

        def body(ci, carry):
            off = base + ci * CHUNK
            pltpu.sync_copy(dst_hbm.at[pl.ds(off, CHUNK)], idx_v)
            pltpu.sync_copy(msg_hbm.at[pl.ds(off, CHUNK)], buf_v)
            pltpu.sync_copy(buf_v, acc.at[idx_v], add=True)
            return carry

        lax.fori_loop(0, N_CHUNKS, body, 0)
        plsc.subcore_barrier()

        pltpu.sync_copy(acc.at[pl.ds(s * ROWS_PER_S, ROWS_PER_S)],
                        out_hbm.at[c, pl.ds(s * ROWS_PER_S, ROWS_PER_S)])

    return k(msg, dst)


# speedup vs baseline: 1.8358x; 1.1170x over previous
import functools

import jax
import jax.numpy as jnp
from jax import lax
from jax.experimental import pallas as pl
from jax.experimental.pallas import tpu as pltpu
from jax.experimental.pallas import tpu_sc as plsc

N = 10000
E = 160000
D_FEAT = 128
CONV_H = 8
N_GRAPHS = 64

NP = 10240
EP = 163840
K = 34
SC = K * CONV_H
BLK_E = 1024
BLK_N = 1024
PAD_DST = 10016


def _make_edge_body(din):
    def _edge_body(ea_ref, xg_ref, w1_ref, b1_ref, wall_ref, r32_ref,
                   bvec_ref, c_ref, msg_ref):
        t = jnp.dot(ea_ref[...], w1_ref[...],
                    preferred_element_type=jnp.float32)
        t = t + b1_ref[...]
        h = jnp.where(t > 0, t, 0.01 * t)
        hexp = jnp.dot(h, r32_ref[...], preferred_element_type=jnp.float32)
        hexp = hexp + bvec_ref[...]
        sg = jnp.dot(xg_ref[:, :din], wall_ref[...],
                     preferred_element_type=jnp.float32)
        msg = jnp.dot(hexp * sg, c_ref[...], preferred_element_type=jnp.float32)
        msg_ref[...] = jnp.pad(msg, ((0, 0), (0, 112)))

    return _edge_body


def _edge_stage(ea, xg, w1, b1, wall, r32, bvec, cmat):
    din = wall.shape[0]
    grid = EP // BLK_E
    return pl.pallas_call(
        _make_edge_body(din),
        grid=(grid,),
        in_specs=[
            pl.BlockSpec((BLK_E, 4), lambda i: (i, 0)),
            pl.BlockSpec((BLK_E, 128), lambda i: (i, 0)),
            pl.BlockSpec((4, 32), lambda i: (0, 0)),
            pl.BlockSpec((1, 32), lambda i: (0, 0)),
            pl.BlockSpec((din, SC), lambda i: (0, 0)),
            pl.BlockSpec((32, SC), lambda i: (0, 0)),
            pl.BlockSpec((1, SC), lambda i: (0, 0)),
            pl.BlockSpec((SC, 16), lambda i: (0, 0)),
        ],
        out_specs=pl.BlockSpec((BLK_E, 128), lambda i: (i, 0)),
        out_shape=jax.ShapeDtypeStruct((EP, 128), jnp.float32),
    )(ea, xg, w1, b1, wall, r32, bvec, cmat)


def _node0_body(p_ref, x_ref, root_ref, bias_ref, h1_ref):
    aggr = p_ref[0, :, :8] + p_ref[1, :, :8]
    r = jnp.dot(x_ref[...], root_ref[...], preferred_element_type=jnp.float32)
    h1 = jnp.maximum(aggr + r + bias_ref[...], 0.0)
    h1_ref[...] = jnp.pad(h1, ((0, 0), (0, 120)))


def _node0_stage(p, x, root, bias):
    grid = NP // BLK_N
    return pl.pallas_call(
        _node0_body,
        grid=(grid,),
        in_specs=[
            pl.BlockSpec((2, BLK_N, 128), lambda i: (0, i, 0)),
            pl.BlockSpec((BLK_N, D_FEAT), lambda i: (i, 0)),
            pl.BlockSpec((D_FEAT, 8), lambda i: (0, 0)),
            pl.BlockSpec((1, 8), lambda i: (0, 0)),
        ],
        out_specs=pl.BlockSpec((BLK_N, 128), lambda i: (i, 0)),
        out_shape=jax.ShapeDtypeStruct((NP, 128), jnp.float32),
    )(p, x, root, bias)


def _node1_body(p_ref, h1_ref, root_ref, bias_ref, batch_ref, pooled_ref):
    i = pl.program_id(0)

    @pl.when(i == 0)
    def _():
        pooled_ref[...] = jnp.zeros_like(pooled_ref)

    aggr = p_ref[0, :, :8] + p_ref[1, :, :8]
    r = jnp.dot(h1_ref[...], root_ref[...], preferred_element_type=jnp.float32)
    h2 = jnp.maximum(aggr + r + bias_ref[...], 0.0)
    g = batch_ref[0, 0, :]
    onehot = (g[None, :] == lax.broadcasted_iota(jnp.int32, (N_GRAPHS, BLK_N), 0)
              ).astype(jnp.float32)
    pooled_ref[...] += jnp.dot(onehot, h2, preferred_element_type=jnp.float32)


def _node1_stage(p, h1, root_pad, bias, batch3d):
    grid = NP // BLK_N
    return pl.pallas_call(
        _node1_body,
        grid=(grid,),
        in_specs=[
            pl.BlockSpec((2, BLK_N, 128), lambda i: (0, i, 0)),
            pl.BlockSpec((BLK_N, 128), lambda i: (i, 0)),
            pl.BlockSpec((128, 8), lambda i: (0, 0)),
            pl.BlockSpec((1, 8), lambda i: (0, 0)),
            pl.BlockSpec((1, 1, BLK_N), lambda i: (i, 0, 0)),
        ],
        out_specs=pl.BlockSpec((N_GRAPHS, 8), lambda i: (0, 0)),
        out_shape=jax.ShapeDtypeStruct((N_GRAPHS, 8), jnp.float32),
    )(p, h1, root_pad, bias, batch3d)


def _head_body(pooled_ref, w0_ref, b0_ref, w1_ref, b1_ref, w2_ref, b2_ref,
               out_ref):
    h = jnp.maximum(jnp.dot(pooled_ref[...], w0_ref[...],
                            preferred_element_type=jnp.float32) + b0_ref[...], 0.0)
    h = jnp.maximum(jnp.dot(h, w1_ref[...],
                            preferred_element_type=jnp.float32) + b1_ref[...], 0.0)
    out_ref[...] = jnp.dot(h, w2_ref[...],
                           preferred_element_type=jnp.float32) + b2_ref[...]


def _head_stage(pooled, w0, b0, w1, b1, w2, b2):
    return pl.pallas_call(
        _head_body,
        out_shape=jax.ShapeDtypeStruct((N_GRAPHS, 12), jnp.float32),
    )(pooled, w0, b0, w1, b1, w2, b2)


def _build_wall(w2, b2, in_ch):
    w2r = jnp.transpose(w2.reshape(32, in_ch, CONV_H), (1, 0, 2))
    wall = jnp.concatenate([
        w2r.reshape(in_ch, 32 * CONV_H),
        b2.reshape(in_ch, CONV_H),
        jnp.zeros((in_ch, CONV_H), jnp.float32),
    ], axis=1)
    return wall


def kernel(x, edge_index, edge_attr, batch, task_id, c0_w1, c0_b1, c0_w2,
           c0_b2, c0_root, c0_bias, c1_w1, c1_b1, c1_w2, c1_b2, c1_root,
           c1_bias, lin0_w, lin0_b, lin1_w, lin1_b, out_w, out_b):
    f32 = jnp.float32
    src = edge_index[0]
    dst = edge_index[1]
    src_p = jnp.pad(src, (0, EP - E))
    dst_p = jnp.pad(dst, (0, EP - E), constant_values=PAD_DST)
    ea_p = jnp.pad(edge_attr, ((0, EP - E), (0, 0)))
    x_p = jnp.pad(x, ((0, NP - N), (0, 0)))
    batch_p = jnp.pad(batch, (0, NP - N), constant_values=127).reshape(
        NP // BLK_N, 1, BLK_N)

    kk = jnp.arange(K * CONV_H, dtype=jnp.int32)
    r32 = (kk[None, :] // CONV_H == jnp.arange(32, dtype=jnp.int32)[:, None]
           ).astype(f32)
    bvec = (kk // CONV_H == 32).astype(f32)[None, :]
    cmat = (kk[:, None] % CONV_H ==
            jnp.arange(16, dtype=jnp.int32)[None, :]).astype(f32)
    cmat = cmat * (jnp.arange(16)[None, :] < 8).astype(f32)

    wall0 = _build_wall(c0_w2, c0_b2, D_FEAT)
    wall1 = jnp.pad(_build_wall(c1_w2, c1_b2, CONV_H), ((0, 8), (0, 0)))
    root1 = jnp.pad(c1_root, ((0, 120), (0, 0)))

    b1_0 = c0_b1.reshape(1, 32)
    b1_1 = c1_b1.reshape(1, 32)
    bias0 = c0_bias.reshape(1, 8)
    bias1 = c1_bias.reshape(1, 8)

    dst2d = dst_p.reshape(EP // CHUNK, CHUNK)

    xg = _sc_gather(x_p, src_p, 128)
    msg0 = _edge_stage(ea_p, xg, c0_w1, b1_0, wall0, r32, bvec, cmat)
    p0 = _sc_scatter(msg0, dst2d)
    h1 = _node0_stage(p0, x_p, c0_root, bias0)

    h1g = _sc_gather(h1, src_p, 128)
    msg1 = _edge_stage(ea_p, h1g, c1_w1, b1_1, wall1, r32, bvec, cmat)
    p1 = _sc_scatter(msg1, dst2d)
    pooled = _node1_stage(p1, h1, root1, bias1, batch_p)

    return _head_stage(pooled, lin0_w, lin0_b.reshape(1, 64), lin1_w,
                       lin1_b.reshape(1, 64), out_w, out_b.reshape(1, 12))


NW = 32
CHUNK = 128
PER_W = EP // NW
N_CHUNKS = PER_W // CHUNK


def _sc_gather(table, idx, out_w):
    d = table.shape[1]
    mesh = plsc.VectorSubcoreMesh(core_axis_name="c", subcore_axis_name="s")

    nbuf = 4

    @functools.partial(
        pl.kernel, mesh=mesh,
        out_type=jax.ShapeDtypeStruct((EP, out_w), jnp.float32),
        scratch_types=[pltpu.VMEM((PER_W,), jnp.int32)]
        + [pltpu.VMEM((CHUNK, d), jnp.float32)] * nbuf
        + [pltpu.SemaphoreType.DMA] * nbuf,
    )
    def k(table_hbm, idx_hbm, out_hbm, idx_v, *bufsem):
        bufs, sems = bufsem[:nbuf], bufsem[nbuf:]
        wid = lax.axis_index("s") * 2 + lax.axis_index("c")
        base = wid * PER_W
        pltpu.sync_copy(idx_hbm.at[pl.ds(base, PER_W)], idx_v)

        def fire(c, j):
            pltpu.make_async_copy(
                table_hbm.at[idx_v.at[pl.ds(c * CHUNK, CHUNK)]], bufs[j],
                sems[j]
            ).start()

        def drain(c, j):
            pltpu.make_async_copy(
                table_hbm.at[idx_v.at[pl.ds(c * CHUNK, CHUNK)]], bufs[j],
                sems[j]
            ).wait()

        def wback(c, j):
            pltpu.sync_copy(bufs[j],
                            out_hbm.at[pl.ds(base + c * CHUNK, CHUNK)])

        for j in range(nbuf - 1):
            fire(j, j)

        def body(i, carry):
            c0 = nbuf * i
            for j in range(nbuf):
                c = c0 + j
                drain(c, j)
                wback(c, j)

                @pl.when(c + nbuf - 1 < N_CHUNKS)
                def _():
                    fire(c + nbuf - 1, (j + nbuf - 1) % nbuf)

            return carry

        lax.fori_loop(0, N_CHUNKS // nbuf, body, 0)

    return k(table, idx)


ROWS_PER_S = NP // 16


def _sc_scatter(msg, dst2d):
    mesh = plsc.VectorSubcoreMesh(core_axis_name="c", subcore_axis_name="s")

    @functools.partial(
        pl.kernel, mesh=mesh,
        out_type=jax.ShapeDtypeStruct((2, NP, 128), jnp.float32),
        scratch_types=[
            pltpu.VMEM((N_CHUNKS, CHUNK), jnp.int32),
            pltpu.VMEM((CHUNK, 128), jnp.float32),
            pltpu.VMEM_SHARED((NP, 128), jnp.float32),
        ],
    )
    def k(msg_hbm, dst_hbm, out_hbm, idx2d, wide, acc):
        c = lax.axis_index("c")
        s = lax.axis_index("s")
        wid = s * 2 + c
        base = wid * PER_W

        pltpu.sync_copy(dst_hbm.at[pl.ds(wid * N_CHUNKS, N_CHUNKS)], idx2d)

        def zvec(i, carry):
            wide[i // 8, pl.ds((i % 8) * 16, 16)] = jnp.zeros((16,),
                                                             jnp.float32)
            return carry

        lax.fori_loop(0, CHUNK * 8, zvec, 0)

        def zcp(j, carry):
            pltpu.sync_copy(wide, acc.at[pl.ds(s * ROWS_PER_S + j * CHUNK,
                                               CHUNK)])
            return carry

        lax.fori_loop(0, ROWS_PER_S // CHUNK, zcp, 0)
        plsc.subcore_barrier()

        def body(ci, carry):
            pltpu.sync_copy(msg_hbm.at[pl.ds(base + ci * CHUNK, CHUNK)], wide)
            pltpu.sync_copy(wide, acc.at[idx2d.at[ci]], add=True)
            return carry

        lax.fori_loop(0, N_CHUNKS, body, 0)
        plsc.subcore_barrier()

        pltpu.sync_copy(acc.at[pl.ds(s * ROWS_PER_S, ROWS_PER_S)],
                        out_hbm.at[c, pl.ds(s * ROWS_PER_S, ROWS_PER_S)])

    return k(msg, dst2d)
